# Initial kernel scaffold; baseline (speedup 1.0000x reference)
import jax
import jax.numpy as jnp
from jax.experimental import pallas as pl


def kernel(x, edge_index, batch, W1_rel, b1, W1_root, ln1_g, ln1_b, W2_rel, b2, W2_root, ln2_g, ln2_b):
    raise NotImplementedError("write your pallas kernel here")


# speedup vs baseline: 8.2014x; 8.2014x over previous
import functools

import jax
import jax.numpy as jnp
from jax import lax
from jax.experimental import pallas as pl
from jax.experimental.pallas import tpu as pltpu
from jax.experimental.pallas import tpu_sc as plsc

N = 100000
E = 1600000
D = 9
G = 64

NC = 2
NS = 16
F = 16

NP = 100352
EP = 1605632
CH = 128
IB = 8
ROWS_PER_TILE = NP // NS
L1_STAGES = EP // (NC * NS) // (CH * IB)
L2_STAGES = EP // NS // (CH * IB)
BLK = 512
NBLK = NP // BLK


def _zero_fill(zbuf):
    def body(i, _):
        zbuf[i, :] = jnp.zeros((F,), jnp.float32)
        return 0
    lax.fori_loop(0, CH, body, 0)


def _agg_body(n_stages, stage_base_fn, idx_sel_fn, out_plane_fn,
              table, src_hbm, dst_hbm, out, src_v, dst_v, rows_v, zbuf, acc, sem):
    c = lax.axis_index("c")
    s = lax.axis_index("s")

    _zero_fill(zbuf)
    row0 = s * ROWS_PER_TILE
    def zero_acc(k, _):
        pltpu.sync_copy(zbuf, acc.at[pl.ds(row0 + k * CH, CH), :])
        return 0
    lax.fori_loop(0, ROWS_PER_TILE // CH, zero_acc, 0)
    plsc.subcore_barrier()

    def stage(st, _):
        rb = stage_base_fn(c, s, st)
        pltpu.sync_copy(idx_sel_fn(src_hbm, c).at[pl.ds(rb, IB), :], src_v)
        pltpu.sync_copy(dst_hbm.at[pl.ds(rb, IB), :], dst_v)
        for j in range(IB):
            pltpu.async_copy(table.at[src_v.at[j]], rows_v.at[j % 2], sem).wait()
            pltpu.sync_copy(rows_v.at[j % 2], acc.at[dst_v.at[j]], add=True)
        return 0
    lax.fori_loop(0, n_stages, stage, 0)
    plsc.subcore_barrier()

    def out_cp(k, _):
        r = row0 + k * CH
        pltpu.sync_copy(acc.at[pl.ds(r, CH), :], rows_v.at[0])
        pltpu.sync_copy(rows_v.at[0], out_plane_fn(out, c).at[pl.ds(r, CH), :])
        return 0
    lax.fori_loop(0, ROWS_PER_TILE // CH, out_cp, 0)


def _make_agg1():
    mesh = plsc.VectorSubcoreMesh(core_axis_name="c", subcore_axis_name="s",
                                  num_cores=NC, num_subcores=NS)
    def stage_base(c, s, st):
        w = s * NC + c
        return w * (L1_STAGES * IB) + st * IB

    body = functools.partial(
        _agg_body, L1_STAGES, stage_base,
        lambda src_hbm, c: src_hbm,
        lambda out, c: out.at[c])

    return pl.kernel(
        body,
        out_type=jax.ShapeDtypeStruct((NC, NP, F), jnp.float32),
        mesh=mesh,
        scratch_types=[
            pltpu.VMEM((IB, CH), jnp.int32),
            pltpu.VMEM((IB, CH), jnp.int32),
            pltpu.VMEM((2, CH, F), jnp.float32),
            pltpu.VMEM((CH, F), jnp.float32),
            pltpu.VMEM_SHARED((NP, F), jnp.float32),
            pltpu.SemaphoreType.DMA,
        ],
        compiler_params=pltpu.CompilerParams(use_tc_tiling_on_sc=False),
    )


def _make_agg2():
    mesh = plsc.VectorSubcoreMesh(core_axis_name="c", subcore_axis_name="s",
                                  num_cores=NC, num_subcores=NS)
    def stage_base(c, s, st):
        return s * (L2_STAGES * IB) + st * IB

    body = functools.partial(
        _agg_body, L2_STAGES, stage_base,
        lambda src_hbm, c: src_hbm.at[c],
        lambda out, c: out.at[c])

    return pl.kernel(
        body,
        out_type=jax.ShapeDtypeStruct((NC, NP, F), jnp.float32),
        mesh=mesh,
        scratch_types=[
            pltpu.VMEM((IB, CH), jnp.int32),
            pltpu.VMEM((IB, CH), jnp.int32),
            pltpu.VMEM((2, CH, F), jnp.float32),
            pltpu.VMEM((CH, F), jnp.float32),
            pltpu.VMEM_SHARED((NP, F), jnp.float32),
            pltpu.SemaphoreType.DMA,
        ],
        compiler_params=pltpu.CompilerParams(use_tc_tiling_on_sc=False),
    )


def _layer1_block(p_ref, x_ref, w_rel_ref, w_root_ref, b_ref, g_ref, be_ref, out_ref):
    i = pl.program_id(0)
    agg = p_ref[0] + p_ref[1]
    xb = x_ref[...]
    dn = (((1,), (1,)), ((), ()))
    h = (lax.dot_general(agg, w_rel_ref[...], dn,
                         preferred_element_type=jnp.float32)
         + lax.dot_general(xb, w_root_ref[...], dn,
                           preferred_element_type=jnp.float32)
         + b_ref[...])
    mu = jnp.mean(h, axis=1, keepdims=True)
    var = jnp.mean((h - mu) ** 2, axis=1, keepdims=True)
    h = (h - mu) / jnp.sqrt(var + 1e-5) * g_ref[...] + be_ref[...]
    h = jnp.maximum(h, 0.0)
    rid = i * BLK + lax.broadcasted_iota(jnp.int32, (BLK, 1), 0)
    h = jnp.where(rid < N, h, 0.0)
    out_ref[0] = h[:, :F]
    out_ref[1] = h[:, F:]


def _layer2_block(a_ref, h_ref, batch_ref, w_rel_ref, w_root_ref, b_ref,
                  g_ref, be_ref, out_ref, acc_ref):
    i = pl.program_id(0)
    agg = jnp.concatenate([a_ref[0], a_ref[1]], axis=1)
    hb = jnp.concatenate([h_ref[0], h_ref[1]], axis=1)
    dn = (((1,), (1,)), ((), ()))
    h2 = (lax.dot_general(agg, w_rel_ref[...], dn,
                          preferred_element_type=jnp.float32)
          + lax.dot_general(hb, w_root_ref[...], dn,
                            preferred_element_type=jnp.float32)
          + b_ref[...])
    mu = jnp.mean(h2, axis=1, keepdims=True)
    var = jnp.mean((h2 - mu) ** 2, axis=1, keepdims=True)
    h2 = (h2 - mu) / jnp.sqrt(var + 1e-5) * g_ref[...] + be_ref[...]
    h2 = jnp.maximum(h2, 0.0)
    bb = batch_ref[0, 0, :]
    onehot = (bb[:, None] == lax.broadcasted_iota(jnp.int32, (BLK, G), 1))
    onehot = onehot.astype(jnp.float32)
    aug = jnp.concatenate([h2, jnp.ones((BLK, G), jnp.float32)], axis=1)
    contrib = lax.dot_general(onehot, aug, (((0,), (0,)), ((), ())),
                              preferred_element_type=jnp.float32)

    @pl.when(i == 0)
    def _():
        acc_ref[...] = jnp.zeros_like(acc_ref)

    acc_ref[...] += contrib

    @pl.when(i == NBLK - 1)
    def _():
        a = acc_ref[...]
        out_ref[...] = a[:, :G] / jnp.maximum(a[:, G:], 1.0)


_layer1_call = pl.pallas_call(
    _layer1_block,
    grid=(NBLK,),
    in_specs=[
        pl.BlockSpec((NC, BLK, F), lambda i: (0, i, 0)),
        pl.BlockSpec((BLK, F), lambda i: (i, 0)),
        pl.BlockSpec((32, F), lambda i: (0, 0)),
        pl.BlockSpec((32, F), lambda i: (0, 0)),
        pl.BlockSpec((1, 32), lambda i: (0, 0)),
        pl.BlockSpec((1, 32), lambda i: (0, 0)),
        pl.BlockSpec((1, 32), lambda i: (0, 0)),
    ],
    out_specs=pl.BlockSpec((NC, BLK, F), lambda i: (0, i, 0)),
    out_shape=jax.ShapeDtypeStruct((NC, NP, F), jnp.float32),
)

_layer2_call = pl.pallas_call(
    _layer2_block,
    grid=(NBLK,),
    in_specs=[
        pl.BlockSpec((NC, BLK, F), lambda i: (0, i, 0)),
        pl.BlockSpec((NC, BLK, F), lambda i: (0, i, 0)),
        pl.BlockSpec((1, 1, BLK), lambda i: (i, 0, 0)),
        pl.BlockSpec((G, 32), lambda i: (0, 0)),
        pl.BlockSpec((G, 32), lambda i: (0, 0)),
        pl.BlockSpec((1, G), lambda i: (0, 0)),
        pl.BlockSpec((1, G), lambda i: (0, 0)),
        pl.BlockSpec((1, G), lambda i: (0, 0)),
    ],
    out_specs=pl.BlockSpec((G, G), lambda i: (0, 0)),
    out_shape=jax.ShapeDtypeStruct((G, G), jnp.float32),
    scratch_shapes=[pltpu.VMEM((G, 2 * G), jnp.float32)],
)


def kernel(x, edge_index, batch, W1_rel, b1, W1_root, ln1_g, ln1_b,
           W2_rel, b2, W2_root, ln2_g, ln2_b):
    x_pad = jnp.zeros((NP, F), jnp.float32).at[:N, :D].set(x)
    src = edge_index[0]
    dst = edge_index[1]
    src_p = jnp.concatenate([src, jnp.full((EP - E,), N, jnp.int32)])
    dst_p = jnp.concatenate([dst, jnp.zeros((EP - E,), jnp.int32)])
    src2d = src_p.reshape(EP // CH, CH)
    dst2d = dst_p.reshape(EP // CH, CH)
    src_l2 = jnp.stack([src2d, src2d + NP])

    w1r = jnp.zeros((32, F), jnp.float32).at[:, :D].set(W1_rel)
    w1o = jnp.zeros((32, F), jnp.float32).at[:, :D].set(W1_root)

    p1 = _make_agg1()(x_pad, src2d, dst2d)
    h_split = _layer1_call(p1, x_pad, w1r, w1o,
                           b1.reshape(1, 32), ln1_g.reshape(1, 32),
                           ln1_b.reshape(1, 32))
    p2 = _make_agg2()(h_split.reshape(NC * NP, F), src_l2, dst2d)

    batch_p = jnp.concatenate([batch, jnp.full((NP - N,), G, jnp.int32)])
    out = _layer2_call(p2, h_split, batch_p.reshape(NBLK, 1, BLK),
                       W2_rel, W2_root, b2.reshape(1, G),
                       ln2_g.reshape(1, G), ln2_b.reshape(1, G))
    return out


# speedup vs baseline: 13.0291x; 1.5887x over previous
import functools

import jax
import jax.numpy as jnp
from jax import lax
from jax.experimental import pallas as pl
from jax.experimental.pallas import tpu as pltpu
from jax.experimental.pallas import tpu_sc as plsc

N = 100000
E = 1600000
D = 9
G = 64

NC = 2
NS = 16
F = 16

NP = 100352
EP = 1605632
CH = 128
IB = 4
ROWS_PER_TILE = NP // NS
L1_STAGES = EP // (NC * NS) // (CH * IB)
L2_STAGES = EP // NS // (CH * IB)
BLK = 512
NBLK = NP // BLK


def _zero_fill(zbuf):
    def body(i, _):
        zbuf[i, :] = jnp.zeros((F,), jnp.float32)
        return 0
    lax.fori_loop(0, CH, body, 0)


def _agg_body(n_stages, stage_base_fn, idx_sel_fn, out_plane_fn,
              table, src_hbm, dst_hbm, out, src_v, dst_v, rows_v, zbuf, acc,
              gsems, ssems):
    c = lax.axis_index("c")
    s = lax.axis_index("s")

    _zero_fill(zbuf)
    row0 = s * ROWS_PER_TILE
    def zero_acc(k, _):
        pltpu.sync_copy(zbuf, acc.at[pl.ds(row0 + k * CH, CH), :])
        return 0
    lax.fori_loop(0, ROWS_PER_TILE // CH, zero_acc, 0)
    plsc.subcore_barrier()

    src_sel = idx_sel_fn(src_hbm, c)

    def load_idx(st, g):
        rb = stage_base_fn(c, s, st)
        pltpu.sync_copy(src_sel.at[pl.ds(rb, IB), :], src_v.at[g])
        pltpu.sync_copy(dst_hbm.at[pl.ds(rb, IB), :], dst_v.at[g])

    def fire_gathers(g):
        for j in range(IB):
            pltpu.async_copy(table.at[src_v.at[g, j]], rows_v.at[g, j],
                             gsems[g])

    def drain_gathers(g):
        for j in range(IB):
            pltpu.make_async_copy(table.at[pl.ds(0, CH), :], rows_v.at[g, j],
                                  gsems[g]).wait()

    def fire_scatters(g):
        for j in range(IB):
            pltpu.async_copy(rows_v.at[g, j], acc.at[dst_v.at[g, j]],
                             ssems[g], add=True)

    def drain_scatters(g):
        for j in range(IB):
            pltpu.make_async_copy(table.at[pl.ds(0, CH), :], rows_v.at[g, j],
                                  ssems[g]).wait()

    n_pairs = n_stages // 2
    load_idx(0, 0)
    fire_gathers(0)

    def stage_pair(p, _):
        e = 2 * p

        @pl.when(p > 0)
        def _():
            drain_scatters(1)
        load_idx(e + 1, 1)
        fire_gathers(1)
        drain_gathers(0)
        fire_scatters(0)

        @pl.when(p + 1 < n_pairs)
        def _():
            drain_scatters(0)
            load_idx(e + 2, 0)
            fire_gathers(0)
        drain_gathers(1)
        fire_scatters(1)
        return 0

    lax.fori_loop(0, n_pairs, stage_pair, 0)
    drain_scatters(0)
    drain_scatters(1)
    plsc.subcore_barrier()

    def out_cp(k, _):
        r = row0 + k * CH
        pltpu.sync_copy(acc.at[pl.ds(r, CH), :], rows_v.at[0, 0])
        pltpu.sync_copy(rows_v.at[0, 0],
                        out_plane_fn(out, c).at[pl.ds(r, CH), :])
        return 0
    lax.fori_loop(0, ROWS_PER_TILE // CH, out_cp, 0)


def _make_agg1():
    mesh = plsc.VectorSubcoreMesh(core_axis_name="c", subcore_axis_name="s",
                                  num_cores=NC, num_subcores=NS)
    def stage_base(c, s, st):
        w = s * NC + c
        return w * (L1_STAGES * IB) + st * IB

    def body(table, src_hbm, dst_hbm, out, src_v, dst_v, rows_v, zbuf, acc,
             gsem0, gsem1, ssem0, ssem1):
        _agg_body(L1_STAGES, stage_base,
                  lambda src_hbm, c: src_hbm,
                  lambda out, c: out.at[c],
                  table, src_hbm, dst_hbm, out, src_v, dst_v, rows_v, zbuf,
                  acc, (gsem0, gsem1), (ssem0, ssem1))

    return pl.kernel(
        body,
        out_type=jax.ShapeDtypeStruct((NC, NP, F), jnp.float32),
        mesh=mesh,
        scratch_types=[
            pltpu.VMEM((2, IB, CH), jnp.int32),
            pltpu.VMEM((2, IB, CH), jnp.int32),
            pltpu.VMEM((2, IB, CH, F), jnp.float32),
            pltpu.VMEM((CH, F), jnp.float32),
            pltpu.VMEM_SHARED((NP, F), jnp.float32),
            pltpu.SemaphoreType.DMA,
            pltpu.SemaphoreType.DMA,
            pltpu.SemaphoreType.DMA,
            pltpu.SemaphoreType.DMA,
        ],
        compiler_params=pltpu.CompilerParams(use_tc_tiling_on_sc=False),
    )


def _make_agg2():
    mesh = plsc.VectorSubcoreMesh(core_axis_name="c", subcore_axis_name="s",
                                  num_cores=NC, num_subcores=NS)
    def stage_base(c, s, st):
        return s * (L2_STAGES * IB) + st * IB

    def body(table, src_hbm, dst_hbm, out, src_v, dst_v, rows_v, zbuf, acc,
             gsem0, gsem1, ssem0, ssem1):
        _agg_body(L2_STAGES, stage_base,
                  lambda src_hbm, c: src_hbm.at[c],
                  lambda out, c: out.at[c],
                  table, src_hbm, dst_hbm, out, src_v, dst_v, rows_v, zbuf,
                  acc, (gsem0, gsem1), (ssem0, ssem1))

    return pl.kernel(
        body,
        out_type=jax.ShapeDtypeStruct((NC, NP, F), jnp.float32),
        mesh=mesh,
        scratch_types=[
            pltpu.VMEM((2, IB, CH), jnp.int32),
            pltpu.VMEM((2, IB, CH), jnp.int32),
            pltpu.VMEM((2, IB, CH, F), jnp.float32),
            pltpu.VMEM((CH, F), jnp.float32),
            pltpu.VMEM_SHARED((NP, F), jnp.float32),
            pltpu.SemaphoreType.DMA,
            pltpu.SemaphoreType.DMA,
            pltpu.SemaphoreType.DMA,
            pltpu.SemaphoreType.DMA,
        ],
        compiler_params=pltpu.CompilerParams(use_tc_tiling_on_sc=False),
    )


def _layer1_block(p_ref, x_ref, w_rel_ref, w_root_ref, b_ref, g_ref, be_ref, out_ref):
    i = pl.program_id(0)
    agg = p_ref[0] + p_ref[1]
    xb = x_ref[...]
    dn = (((1,), (1,)), ((), ()))
    h = (lax.dot_general(agg, w_rel_ref[...], dn,
                         preferred_element_type=jnp.float32)
         + lax.dot_general(xb, w_root_ref[...], dn,
                           preferred_element_type=jnp.float32)
         + b_ref[...])
    mu = jnp.mean(h, axis=1, keepdims=True)
    var = jnp.mean((h - mu) ** 2, axis=1, keepdims=True)
    h = (h - mu) / jnp.sqrt(var + 1e-5) * g_ref[...] + be_ref[...]
    h = jnp.maximum(h, 0.0)
    rid = i * BLK + lax.broadcasted_iota(jnp.int32, (BLK, 1), 0)
    h = jnp.where(rid < N, h, 0.0)
    out_ref[0] = h[:, :F]
    out_ref[1] = h[:, F:]


def _layer2_block(a_ref, h_ref, batch_ref, w_rel_ref, w_root_ref, b_ref,
                  g_ref, be_ref, out_ref, acc_ref):
    i = pl.program_id(0)
    agg = jnp.concatenate([a_ref[0], a_ref[1]], axis=1)
    hb = jnp.concatenate([h_ref[0], h_ref[1]], axis=1)
    dn = (((1,), (1,)), ((), ()))
    h2 = (lax.dot_general(agg, w_rel_ref[...], dn,
                          preferred_element_type=jnp.float32)
          + lax.dot_general(hb, w_root_ref[...], dn,
                            preferred_element_type=jnp.float32)
          + b_ref[...])
    mu = jnp.mean(h2, axis=1, keepdims=True)
    var = jnp.mean((h2 - mu) ** 2, axis=1, keepdims=True)
    h2 = (h2 - mu) / jnp.sqrt(var + 1e-5) * g_ref[...] + be_ref[...]
    h2 = jnp.maximum(h2, 0.0)
    bb = batch_ref[0, 0, :]
    onehot = (bb[:, None] == lax.broadcasted_iota(jnp.int32, (BLK, G), 1))
    onehot = onehot.astype(jnp.float32)
    aug = jnp.concatenate([h2, jnp.ones((BLK, G), jnp.float32)], axis=1)
    contrib = lax.dot_general(onehot, aug, (((0,), (0,)), ((), ())),
                              preferred_element_type=jnp.float32)

    @pl.when(i == 0)
    def _():
        acc_ref[...] = jnp.zeros_like(acc_ref)

    acc_ref[...] += contrib

    @pl.when(i == NBLK - 1)
    def _():
        a = acc_ref[...]
        out_ref[...] = a[:, :G] / jnp.maximum(a[:, G:], 1.0)


_layer1_call = pl.pallas_call(
    _layer1_block,
    grid=(NBLK,),
    in_specs=[
        pl.BlockSpec((NC, BLK, F), lambda i: (0, i, 0)),
        pl.BlockSpec((BLK, F), lambda i: (i, 0)),
        pl.BlockSpec((32, F), lambda i: (0, 0)),
        pl.BlockSpec((32, F), lambda i: (0, 0)),
        pl.BlockSpec((1, 32), lambda i: (0, 0)),
        pl.BlockSpec((1, 32), lambda i: (0, 0)),
        pl.BlockSpec((1, 32), lambda i: (0, 0)),
    ],
    out_specs=pl.BlockSpec((NC, BLK, F), lambda i: (0, i, 0)),
    out_shape=jax.ShapeDtypeStruct((NC, NP, F), jnp.float32),
)

_layer2_call = pl.pallas_call(
    _layer2_block,
    grid=(NBLK,),
    in_specs=[
        pl.BlockSpec((NC, BLK, F), lambda i: (0, i, 0)),
        pl.BlockSpec((NC, BLK, F), lambda i: (0, i, 0)),
        pl.BlockSpec((1, 1, BLK), lambda i: (i, 0, 0)),
        pl.BlockSpec((G, 32), lambda i: (0, 0)),
        pl.BlockSpec((G, 32), lambda i: (0, 0)),
        pl.BlockSpec((1, G), lambda i: (0, 0)),
        pl.BlockSpec((1, G), lambda i: (0, 0)),
        pl.BlockSpec((1, G), lambda i: (0, 0)),
    ],
    out_specs=pl.BlockSpec((G, G), lambda i: (0, 0)),
    out_shape=jax.ShapeDtypeStruct((G, G), jnp.float32),
    scratch_shapes=[pltpu.VMEM((G, 2 * G), jnp.float32)],
)


def kernel(x, edge_index, batch, W1_rel, b1, W1_root, ln1_g, ln1_b,
           W2_rel, b2, W2_root, ln2_g, ln2_b):
    x_pad = jnp.zeros((NP, F), jnp.float32).at[:N, :D].set(x)
    src = edge_index[0]
    dst = edge_index[1]
    src_p = jnp.concatenate([src, jnp.full((EP - E,), N, jnp.int32)])
    dst_p = jnp.concatenate([dst, jnp.zeros((EP - E,), jnp.int32)])
    src2d = src_p.reshape(EP // CH, CH)
    dst2d = dst_p.reshape(EP // CH, CH)
    src_l2 = jnp.stack([src2d, src2d + NP])

    w1r = jnp.zeros((32, F), jnp.float32).at[:, :D].set(W1_rel)
    w1o = jnp.zeros((32, F), jnp.float32).at[:, :D].set(W1_root)

    p1 = _make_agg1()(x_pad, src2d, dst2d)
    h_split = _layer1_call(p1, x_pad, w1r, w1o,
                           b1.reshape(1, 32), ln1_g.reshape(1, 32),
                           ln1_b.reshape(1, 32))
    p2 = _make_agg2()(h_split.reshape(NC * NP, F), src_l2, dst2d)

    batch_p = jnp.concatenate([batch, jnp.full((NP - N,), G, jnp.int32)])
    out = _layer2_call(p2, h_split, batch_p.reshape(NBLK, 1, BLK),
                       W2_rel, W2_root, b2.reshape(1, G),
                       ln2_g.reshape(1, G), ln2_b.reshape(1, G))
    return out


# speedup vs baseline: 15.8388x; 1.2157x over previous
import functools

import jax
import jax.numpy as jnp
from jax import lax
from jax.experimental import pallas as pl
from jax.experimental.pallas import tpu as pltpu
from jax.experimental.pallas import tpu_sc as plsc

N = 100000
E = 1600000
D = 9
G = 64

NC = 2
NS = 16
F = 16

NP = 100352
EP = 1605632
CH = 128
IB = 4
ROWS_PER_TILE = NP // NS
L1_STAGES = EP // (NC * NS) // (CH * IB)
L2_STAGES = EP // NS // (CH * IB)
BLK = 1024
NBLK = NP // BLK


GRP = IB * CH


def _agg_body(n_stages, stage_base_fn, gather_fn, out_plane_fn,
              dummy_hbm, edges_il, out, idx_v, rows_v, acc, gsems, ssems):
    c = lax.axis_index("c")
    s = lax.axis_index("s")
    row0 = s * ROWS_PER_TILE

    def zfill(i, _):
        rows_v[pl.ds(i * 16, 16), :] = jnp.zeros((16, F), jnp.float32)
        return 0
    lax.fori_loop(0, GRP // 16, zfill, 0)
    def zero_acc(k, _):
        pltpu.sync_copy(rows_v.at[pl.ds(0, GRP), :],
                        acc.at[pl.ds(row0 + k * GRP, GRP), :])
        return 0
    lax.fori_loop(0, ROWS_PER_TILE // GRP, zero_acc, 0)
    pltpu.sync_copy(rows_v.at[pl.ds(0, CH), :],
                    acc.at[pl.ds(row0 + (ROWS_PER_TILE // GRP) * GRP,
                                 ROWS_PER_TILE - (ROWS_PER_TILE // GRP) * GRP),
                           :])
    plsc.subcore_barrier()

    def load_idx(st, g):
        n = stage_base_fn(c, s, st)
        pltpu.sync_copy(edges_il.at[n], idx_v.at[g])

    def drain_group(g, sems):
        pltpu.make_async_copy(dummy_hbm.at[pl.ds(0, GRP), :],
                              rows_v.at[pl.ds(g * GRP, GRP), :],
                              sems[g]).wait()

    def fire_scatters(g):
        for j in range(IB):
            pltpu.async_copy(rows_v.at[pl.ds(g * GRP + j * CH, CH), :],
                             acc.at[idx_v.at[g, 1, j]], ssems[g], add=True)

    n_pairs = n_stages // 2
    load_idx(0, 0)
    gather_fn(0)

    def stage_pair(p, _):
        e = 2 * p

        @pl.when(p > 0)
        def _():
            drain_group(1, ssems)
        load_idx(e + 1, 1)
        gather_fn(1)
        drain_group(0, gsems)
        fire_scatters(0)

        @pl.when(p + 1 < n_pairs)
        def _():
            drain_group(0, ssems)
            load_idx(e + 2, 0)
            gather_fn(0)
        drain_group(1, gsems)
        fire_scatters(1)
        return 0

    lax.fori_loop(0, n_pairs, stage_pair, 0)
    drain_group(0, ssems)
    drain_group(1, ssems)
    plsc.subcore_barrier()

    out_pl = out_plane_fn(out, c)
    n_out = ROWS_PER_TILE // GRP
    tail = ROWS_PER_TILE - n_out * GRP

    def out_cp(k2, _):
        for g in (0, 1):
            r = row0 + (2 * k2 + g) * GRP

            @pl.when(k2 > 0)
            def _(g=g):
                drain_group(g, gsems)
            pltpu.sync_copy(acc.at[pl.ds(r, GRP), :],
                            rows_v.at[pl.ds(g * GRP, GRP), :])
            pltpu.async_copy(rows_v.at[pl.ds(g * GRP, GRP), :],
                             out_pl.at[pl.ds(r, GRP), :], gsems[g])
        return 0
    lax.fori_loop(0, n_out // 2, out_cp, 0)
    rt = row0 + n_out * GRP
    gt = n_out % 2
    drain_group(gt, gsems)
    pltpu.sync_copy(acc.at[pl.ds(rt, tail), :],
                    rows_v.at[pl.ds(gt * GRP, tail), :])
    pltpu.async_copy(rows_v.at[pl.ds(gt * GRP, tail), :],
                     out_pl.at[pl.ds(rt, tail), :], gsems[gt])
    pltpu.make_async_copy(dummy_hbm.at[pl.ds(0, tail), :],
                          rows_v.at[pl.ds(gt * GRP, tail), :],
                          gsems[gt]).wait()
    pltpu.make_async_copy(dummy_hbm.at[pl.ds(0, GRP), :],
                          rows_v.at[pl.ds((1 - gt) * GRP, GRP), :],
                          gsems[1 - gt]).wait()


_SC_SCRATCH = [
    pltpu.VMEM((2, 2, IB, CH), jnp.int32),
    pltpu.VMEM((2 * GRP, F), jnp.float32),
    pltpu.VMEM_SHARED((NP, F), jnp.float32),
    pltpu.SemaphoreType.DMA,
    pltpu.SemaphoreType.DMA,
    pltpu.SemaphoreType.DMA,
    pltpu.SemaphoreType.DMA,
]


def _sc_mesh():
    return plsc.VectorSubcoreMesh(core_axis_name="c", subcore_axis_name="s",
                                  num_cores=NC, num_subcores=NS)


def _make_agg1():
    def stage_base(c, s, st):
        return (s * NC + c) * L1_STAGES + st

    def body(table, edges_il, out, idx_v, rows_v, acc,
             gsem0, gsem1, ssem0, ssem1):
        gsems = (gsem0, gsem1)

        def gather_fn(g):
            for j in range(IB):
                pltpu.async_copy(table.at[idx_v.at[g, 0, j]],
                                 rows_v.at[pl.ds(g * GRP + j * CH, CH), :],
                                 gsems[g])

        _agg_body(L1_STAGES, stage_base, gather_fn,
                  lambda out, c: out.at[c],
                  table, edges_il, out, idx_v, rows_v, acc,
                  gsems, (ssem0, ssem1))

    return pl.kernel(
        body,
        out_type=jax.ShapeDtypeStruct((NC, NP, F), jnp.float32),
        mesh=_sc_mesh(),
        scratch_types=list(_SC_SCRATCH),
        compiler_params=pltpu.CompilerParams(use_tc_tiling_on_sc=False),
    )


def _make_agg2():
    def stage_base(c, s, st):
        return s * L2_STAGES + st

    def body(tab0, tab1, edges_il, out, idx_v, rows_v, acc,
             gsem0, gsem1, ssem0, ssem1):
        gsems = (gsem0, gsem1)

        def gather_fn(g):
            c = lax.axis_index("c")

            @pl.when(c == 0)
            def _():
                for j in range(IB):
                    pltpu.async_copy(tab0.at[idx_v.at[g, 0, j]],
                                     rows_v.at[pl.ds(g * GRP + j * CH, CH), :],
                                     gsems[g])

            @pl.when(c == 1)
            def _():
                for j in range(IB):
                    pltpu.async_copy(tab1.at[idx_v.at[g, 0, j]],
                                     rows_v.at[pl.ds(g * GRP + j * CH, CH), :],
                                     gsems[g])

        _agg_body(L2_STAGES, stage_base, gather_fn,
                  lambda out, c: out.at[c],
                  tab0, edges_il, out, idx_v, rows_v, acc,
                  gsems, (ssem0, ssem1))

    return pl.kernel(
        body,
        out_type=jax.ShapeDtypeStruct((NC, NP, F), jnp.float32),
        mesh=_sc_mesh(),
        scratch_types=list(_SC_SCRATCH),
        compiler_params=pltpu.CompilerParams(use_tc_tiling_on_sc=False),
    )


def _layer1_block(p_ref, x_ref, w_rel_ref, w_root_ref, b_ref, g_ref, be_ref,
                  out0_ref, out1_ref):
    i = pl.program_id(0)
    agg = p_ref[0] + p_ref[1]
    xb = x_ref[...]
    dn = (((1,), (1,)), ((), ()))
    h = (lax.dot_general(agg, w_rel_ref[...], dn,
                         preferred_element_type=jnp.float32)
         + lax.dot_general(xb, w_root_ref[...], dn,
                           preferred_element_type=jnp.float32)
         + b_ref[...])
    mu = jnp.mean(h, axis=1, keepdims=True)
    var = jnp.mean((h - mu) ** 2, axis=1, keepdims=True)
    h = (h - mu) / jnp.sqrt(var + 1e-5) * g_ref[...] + be_ref[...]
    h = jnp.maximum(h, 0.0)
    rid = i * BLK + lax.broadcasted_iota(jnp.int32, (BLK, 1), 0)
    h = jnp.where(rid < N, h, 0.0)
    out0_ref[...] = h[:, :F]
    out1_ref[...] = h[:, F:]


def _layer2_block(a_ref, h0_ref, h1_ref, batch_ref, w_rel_ref, w_root_ref,
                  b_ref, g_ref, be_ref, out_ref, acc_ref):
    i = pl.program_id(0)
    agg = jnp.concatenate([a_ref[0], a_ref[1]], axis=1)
    hb = jnp.concatenate([h0_ref[...], h1_ref[...]], axis=1)
    dn = (((1,), (1,)), ((), ()))
    h2 = (lax.dot_general(agg, w_rel_ref[...], dn,
                          preferred_element_type=jnp.float32)
          + lax.dot_general(hb, w_root_ref[...], dn,
                            preferred_element_type=jnp.float32)
          + b_ref[...])
    mu = jnp.mean(h2, axis=1, keepdims=True)
    var = jnp.mean((h2 - mu) ** 2, axis=1, keepdims=True)
    h2 = (h2 - mu) / jnp.sqrt(var + 1e-5) * g_ref[...] + be_ref[...]
    h2 = jnp.maximum(h2, 0.0)
    bb = batch_ref[0, 0, :]
    onehot = (bb[:, None] == lax.broadcasted_iota(jnp.int32, (BLK, G), 1))
    onehot = onehot.astype(jnp.float32)
    aug = jnp.concatenate([h2, jnp.ones((BLK, G), jnp.float32)], axis=1)
    contrib = lax.dot_general(onehot, aug, (((0,), (0,)), ((), ())),
                              preferred_element_type=jnp.float32)

    @pl.when(i == 0)
    def _():
        acc_ref[...] = jnp.zeros_like(acc_ref)

    acc_ref[...] += contrib

    @pl.when(i == NBLK - 1)
    def _():
        a = acc_ref[...]
        out_ref[...] = a[:, :G] / jnp.maximum(a[:, G:], 1.0)


_layer1_call = pl.pallas_call(
    _layer1_block,
    grid=(NBLK,),
    in_specs=[
        pl.BlockSpec((NC, BLK, F), lambda i: (0, i, 0)),
        pl.BlockSpec((BLK, F), lambda i: (i, 0)),
        pl.BlockSpec((32, F), lambda i: (0, 0)),
        pl.BlockSpec((32, F), lambda i: (0, 0)),
        pl.BlockSpec((1, 32), lambda i: (0, 0)),
        pl.BlockSpec((1, 32), lambda i: (0, 0)),
        pl.BlockSpec((1, 32), lambda i: (0, 0)),
    ],
    out_specs=[pl.BlockSpec((BLK, F), lambda i: (i, 0)),
               pl.BlockSpec((BLK, F), lambda i: (i, 0))],
    out_shape=[jax.ShapeDtypeStruct((NP, F), jnp.float32),
               jax.ShapeDtypeStruct((NP, F), jnp.float32)],
)

_layer2_call = pl.pallas_call(
    _layer2_block,
    grid=(NBLK,),
    in_specs=[
        pl.BlockSpec((NC, BLK, F), lambda i: (0, i, 0)),
        pl.BlockSpec((BLK, F), lambda i: (i, 0)),
        pl.BlockSpec((BLK, F), lambda i: (i, 0)),
        pl.BlockSpec((1, 1, BLK), lambda i: (i, 0, 0)),
        pl.BlockSpec((G, 32), lambda i: (0, 0)),
        pl.BlockSpec((G, 32), lambda i: (0, 0)),
        pl.BlockSpec((1, G), lambda i: (0, 0)),
        pl.BlockSpec((1, G), lambda i: (0, 0)),
        pl.BlockSpec((1, G), lambda i: (0, 0)),
    ],
    out_specs=pl.BlockSpec((G, G), lambda i: (0, 0)),
    out_shape=jax.ShapeDtypeStruct((G, G), jnp.float32),
    scratch_shapes=[pltpu.VMEM((G, 2 * G), jnp.float32)],
)


def kernel(x, edge_index, batch, W1_rel, b1, W1_root, ln1_g, ln1_b,
           W2_rel, b2, W2_root, ln2_g, ln2_b):
    x_pad = jnp.zeros((NP, F), jnp.float32).at[:N, :D].set(x)
    src = edge_index[0]
    dst = edge_index[1]
    src_p = jnp.concatenate([src, jnp.full((EP - E,), N, jnp.int32)])
    dst_p = jnp.concatenate([dst, jnp.zeros((EP - E,), jnp.int32)])
    edges_il = jnp.stack([src_p.reshape(EP // GRP, IB, CH),
                          dst_p.reshape(EP // GRP, IB, CH)], axis=1)

    w1r = jnp.zeros((32, F), jnp.float32).at[:, :D].set(W1_rel)
    w1o = jnp.zeros((32, F), jnp.float32).at[:, :D].set(W1_root)

    p1 = _make_agg1()(x_pad, edges_il)
    h0, h1 = _layer1_call(p1, x_pad, w1r, w1o,
                          b1.reshape(1, 32), ln1_g.reshape(1, 32),
                          ln1_b.reshape(1, 32))
    p2 = _make_agg2()(h0, h1, edges_il)

    batch_p = jnp.concatenate([batch, jnp.full((NP - N,), G, jnp.int32)])
    out = _layer2_call(p2, h0, h1, batch_p.reshape(NBLK, 1, BLK),
                       W2_rel, W2_root, b2.reshape(1, G),
                       ln2_g.reshape(1, G), ln2_b.reshape(1, G))
    return out


# speedup vs baseline: 18.6539x; 1.1777x over previous
import functools

import jax
import jax.numpy as jnp
from jax import lax
from jax.experimental import pallas as pl
from jax.experimental.pallas import tpu as pltpu
from jax.experimental.pallas import tpu_sc as plsc

N = 100000
E = 1600000
D = 9
G = 64

NC = 2
NS = 16
F = 16
F2 = 32

NP = 100352
EP = 1605632
CH = 128
IB = 4
ROWS_PER_TILE = NP // NS
L1_STAGES = EP // (NC * NS) // (CH * IB)
BLK = 1024
NBLK = NP // BLK


GRP = IB * CH


def _agg_body(n_stages, stage_base_fn, gather_fn, out_plane_fn,
              dummy_hbm, edges_il, out, idx_v, rows_v, acc, gsems, ssems,
              feat, dtype):
    c = lax.axis_index("c")
    s = lax.axis_index("s")
    row0 = s * ROWS_PER_TILE

    def zfill(i, _):
        rows_v[pl.ds(i * 16, 16), :] = jnp.zeros((16, feat), dtype)
        return 0
    lax.fori_loop(0, GRP // 16, zfill, 0)
    def zero_acc(k, _):
        pltpu.sync_copy(rows_v.at[pl.ds(0, GRP), :],
                        acc.at[pl.ds(row0 + k * GRP, GRP), :])
        return 0
    lax.fori_loop(0, ROWS_PER_TILE // GRP, zero_acc, 0)
    pltpu.sync_copy(rows_v.at[pl.ds(0, CH), :],
                    acc.at[pl.ds(row0 + (ROWS_PER_TILE // GRP) * GRP,
                                 ROWS_PER_TILE - (ROWS_PER_TILE // GRP) * GRP),
                           :])
    plsc.subcore_barrier()

    def load_idx(st, g):
        n = stage_base_fn(c, s, st)
        pltpu.sync_copy(edges_il.at[n], idx_v.at[g])

    def drain_group(g, sems):
        pltpu.make_async_copy(dummy_hbm.at[pl.ds(0, GRP), :],
                              rows_v.at[pl.ds(g * GRP, GRP), :],
                              sems[g]).wait()

    def fire_scatters(g):
        for j in range(IB):
            pltpu.async_copy(rows_v.at[pl.ds(g * GRP + j * CH, CH), :],
                             acc.at[idx_v.at[g, 1, j]], ssems[g], add=True)

    n_pairs = n_stages // 2
    load_idx(0, 0)
    gather_fn(0)

    def stage_pair(p, _):
        e = 2 * p

        @pl.when(p > 0)
        def _():
            drain_group(1, ssems)
        load_idx(e + 1, 1)
        gather_fn(1)
        drain_group(0, gsems)
        fire_scatters(0)

        @pl.when(p + 1 < n_pairs)
        def _():
            drain_group(0, ssems)
            load_idx(e + 2, 0)
            gather_fn(0)
        drain_group(1, gsems)
        fire_scatters(1)
        return 0

    lax.fori_loop(0, n_pairs, stage_pair, 0)
    drain_group(0, ssems)
    drain_group(1, ssems)
    plsc.subcore_barrier()

    out_pl = out_plane_fn(out, c)
    n_out = ROWS_PER_TILE // GRP
    tail = ROWS_PER_TILE - n_out * GRP

    def out_cp(k2, _):
        for g in (0, 1):
            r = row0 + (2 * k2 + g) * GRP

            @pl.when(k2 > 0)
            def _(g=g):
                drain_group(g, gsems)
            pltpu.sync_copy(acc.at[pl.ds(r, GRP), :],
                            rows_v.at[pl.ds(g * GRP, GRP), :])
            pltpu.async_copy(rows_v.at[pl.ds(g * GRP, GRP), :],
                             out_pl.at[pl.ds(r, GRP), :], gsems[g])
        return 0
    lax.fori_loop(0, n_out // 2, out_cp, 0)
    rt = row0 + n_out * GRP
    gt = n_out % 2
    drain_group(gt, gsems)
    pltpu.sync_copy(acc.at[pl.ds(rt, tail), :],
                    rows_v.at[pl.ds(gt * GRP, tail), :])
    pltpu.async_copy(rows_v.at[pl.ds(gt * GRP, tail), :],
                     out_pl.at[pl.ds(rt, tail), :], gsems[gt])
    pltpu.make_async_copy(dummy_hbm.at[pl.ds(0, tail), :],
                          rows_v.at[pl.ds(gt * GRP, tail), :],
                          gsems[gt]).wait()
    pltpu.make_async_copy(dummy_hbm.at[pl.ds(0, GRP), :],
                          rows_v.at[pl.ds((1 - gt) * GRP, GRP), :],
                          gsems[1 - gt]).wait()


_SC_SCRATCH = [
    pltpu.VMEM((2, 2, IB, CH), jnp.int32),
    pltpu.VMEM((2 * GRP, F), jnp.float32),
    pltpu.VMEM_SHARED((NP, F), jnp.float32),
    pltpu.SemaphoreType.DMA,
    pltpu.SemaphoreType.DMA,
    pltpu.SemaphoreType.DMA,
    pltpu.SemaphoreType.DMA,
]


def _sc_mesh():
    return plsc.VectorSubcoreMesh(core_axis_name="c", subcore_axis_name="s",
                                  num_cores=NC, num_subcores=NS)


def _make_agg1():
    def stage_base(c, s, st):
        return (s * NC + c) * L1_STAGES + st

    def body(table, edges_il, out, idx_v, rows_v, acc,
             gsem0, gsem1, ssem0, ssem1):
        gsems = (gsem0, gsem1)

        def gather_fn(g):
            for j in range(IB):
                pltpu.async_copy(table.at[idx_v.at[g, 0, j]],
                                 rows_v.at[pl.ds(g * GRP + j * CH, CH), :],
                                 gsems[g])

        _agg_body(L1_STAGES, stage_base, gather_fn,
                  lambda out, c: out.at[c],
                  table, edges_il, out, idx_v, rows_v, acc,
                  gsems, (ssem0, ssem1), F, jnp.float32)

    return pl.kernel(
        body,
        out_type=jax.ShapeDtypeStruct((NC, NP, F), jnp.float32),
        mesh=_sc_mesh(),
        scratch_types=list(_SC_SCRATCH),
        compiler_params=pltpu.CompilerParams(use_tc_tiling_on_sc=False),
    )


def _make_agg2():
    def stage_base(c, s, st):
        return (s * NC + c) * L1_STAGES + st

    def body(table, edges_il, out, idx_v, rows_v, acc,
             gsem0, gsem1, ssem0, ssem1):
        gsems = (gsem0, gsem1)

        def gather_fn(g):
            for j in range(IB):
                pltpu.async_copy(table.at[idx_v.at[g, 0, j]],
                                 rows_v.at[pl.ds(g * GRP + j * CH, CH), :],
                                 gsems[g])

        _agg_body(L1_STAGES, stage_base, gather_fn,
                  lambda out, c: out.at[c],
                  table, edges_il, out, idx_v, rows_v, acc,
                  gsems, (ssem0, ssem1), F2, jnp.bfloat16)

    return pl.kernel(
        body,
        out_type=jax.ShapeDtypeStruct((NC, NP, F2), jnp.bfloat16),
        mesh=_sc_mesh(),
        scratch_types=[
            pltpu.VMEM((2, 2, IB, CH), jnp.int32),
            pltpu.VMEM((2 * GRP, F2), jnp.bfloat16),
            pltpu.VMEM_SHARED((NP, F2), jnp.bfloat16),
            pltpu.SemaphoreType.DMA,
            pltpu.SemaphoreType.DMA,
            pltpu.SemaphoreType.DMA,
            pltpu.SemaphoreType.DMA,
        ],
        compiler_params=pltpu.CompilerParams(use_tc_tiling_on_sc=False),
    )


def _layer1_block(p_ref, x_ref, w_rel_ref, w_root_ref, b_ref, g_ref, be_ref,
                  out_ref):
    i = pl.program_id(0)
    agg = p_ref[0] + p_ref[1]
    xb = x_ref[...]
    dn = (((1,), (1,)), ((), ()))
    h = (lax.dot_general(agg, w_rel_ref[...], dn,
                         preferred_element_type=jnp.float32)
         + lax.dot_general(xb, w_root_ref[...], dn,
                           preferred_element_type=jnp.float32)
         + b_ref[...])
    mu = jnp.mean(h, axis=1, keepdims=True)
    var = jnp.mean((h - mu) ** 2, axis=1, keepdims=True)
    h = (h - mu) / jnp.sqrt(var + 1e-5) * g_ref[...] + be_ref[...]
    h = jnp.maximum(h, 0.0)
    rid = i * BLK + lax.broadcasted_iota(jnp.int32, (BLK, 1), 0)
    h = jnp.where(rid < N, h, 0.0)
    out_ref[...] = h.astype(jnp.bfloat16)


def _layer2_block(a_ref, h_ref, batch_ref, w_rel_ref, w_root_ref,
                  b_ref, g_ref, be_ref, out_ref, acc_ref):
    i = pl.program_id(0)
    agg = (a_ref[0].astype(jnp.float32) + a_ref[1].astype(jnp.float32))
    hb = h_ref[...].astype(jnp.float32)
    dn = (((1,), (1,)), ((), ()))
    h2 = (lax.dot_general(agg, w_rel_ref[...], dn,
                          preferred_element_type=jnp.float32)
          + lax.dot_general(hb, w_root_ref[...], dn,
                            preferred_element_type=jnp.float32)
          + b_ref[...])
    mu = jnp.mean(h2, axis=1, keepdims=True)
    var = jnp.mean((h2 - mu) ** 2, axis=1, keepdims=True)
    h2 = (h2 - mu) / jnp.sqrt(var + 1e-5) * g_ref[...] + be_ref[...]
    h2 = jnp.maximum(h2, 0.0)
    bb = batch_ref[0, 0, :]
    onehot = (bb[:, None] == lax.broadcasted_iota(jnp.int32, (BLK, G), 1))
    onehot = onehot.astype(jnp.float32)
    aug = jnp.concatenate([h2, jnp.ones((BLK, G), jnp.float32)], axis=1)
    contrib = lax.dot_general(onehot, aug, (((0,), (0,)), ((), ())),
                              preferred_element_type=jnp.float32)

    @pl.when(i == 0)
    def _():
        acc_ref[...] = jnp.zeros_like(acc_ref)

    acc_ref[...] += contrib

    @pl.when(i == NBLK - 1)
    def _():
        a = acc_ref[...]
        out_ref[...] = a[:, :G] / jnp.maximum(a[:, G:], 1.0)


_layer1_call = pl.pallas_call(
    _layer1_block,
    grid=(NBLK,),
    in_specs=[
        pl.BlockSpec((NC, BLK, F), lambda i: (0, i, 0)),
        pl.BlockSpec((BLK, F), lambda i: (i, 0)),
        pl.BlockSpec((32, F), lambda i: (0, 0)),
        pl.BlockSpec((32, F), lambda i: (0, 0)),
        pl.BlockSpec((1, 32), lambda i: (0, 0)),
        pl.BlockSpec((1, 32), lambda i: (0, 0)),
        pl.BlockSpec((1, 32), lambda i: (0, 0)),
    ],
    out_specs=pl.BlockSpec((BLK, F2), lambda i: (i, 0)),
    out_shape=jax.ShapeDtypeStruct((NP, F2), jnp.bfloat16),
)

_layer2_call = pl.pallas_call(
    _layer2_block,
    grid=(NBLK,),
    in_specs=[
        pl.BlockSpec((NC, BLK, F2), lambda i: (0, i, 0)),
        pl.BlockSpec((BLK, F2), lambda i: (i, 0)),
        pl.BlockSpec((1, 1, BLK), lambda i: (i, 0, 0)),
        pl.BlockSpec((G, 32), lambda i: (0, 0)),
        pl.BlockSpec((G, 32), lambda i: (0, 0)),
        pl.BlockSpec((1, G), lambda i: (0, 0)),
        pl.BlockSpec((1, G), lambda i: (0, 0)),
        pl.BlockSpec((1, G), lambda i: (0, 0)),
    ],
    out_specs=pl.BlockSpec((G, G), lambda i: (0, 0)),
    out_shape=jax.ShapeDtypeStruct((G, G), jnp.float32),
    scratch_shapes=[pltpu.VMEM((G, 2 * G), jnp.float32)],
)


def kernel(x, edge_index, batch, W1_rel, b1, W1_root, ln1_g, ln1_b,
           W2_rel, b2, W2_root, ln2_g, ln2_b):
    x_pad = jnp.zeros((NP, F), jnp.float32).at[:N, :D].set(x)
    src = edge_index[0]
    dst = edge_index[1]
    src_p = jnp.concatenate([src, jnp.full((EP - E,), N, jnp.int32)])
    dst_p = jnp.concatenate([dst, jnp.zeros((EP - E,), jnp.int32)])
    edges_il = jnp.stack([src_p.reshape(EP // GRP, IB, CH),
                          dst_p.reshape(EP // GRP, IB, CH)], axis=1)

    w1r = jnp.zeros((32, F), jnp.float32).at[:, :D].set(W1_rel)
    w1o = jnp.zeros((32, F), jnp.float32).at[:, :D].set(W1_root)

    p1 = _make_agg1()(x_pad, edges_il)
    h_bf = _layer1_call(p1, x_pad, w1r, w1o,
                        b1.reshape(1, 32), ln1_g.reshape(1, 32),
                        ln1_b.reshape(1, 32))
    p2 = _make_agg2()(h_bf, edges_il)

    batch_p = jnp.concatenate([batch, jnp.full((NP - N,), G, jnp.int32)])
    out = _layer2_call(p2, h_bf, batch_p.reshape(NBLK, 1, BLK),
                       W2_rel, W2_root, b2.reshape(1, G),
                       ln2_g.reshape(1, G), ln2_b.reshape(1, G))
    return out


# speedup vs baseline: 22.1300x; 1.1864x over previous
import functools

import jax
import jax.numpy as jnp
from jax import lax
from jax.experimental import pallas as pl
from jax.experimental.pallas import tpu as pltpu
from jax.experimental.pallas import tpu_sc as plsc

N = 100000
E = 1600000
D = 9
G = 64

NC = 2
NS = 16
F = 16
F2 = 32

NP = 100352
EP = 1605632
CH = 128
IB = 4
ROWS_PER_TILE = NP // NS
L1_STAGES = EP // (NC * NS) // (CH * IB)
BLK = 1024
NBLK = NP // BLK


GRP = IB * CH


def _agg_body(n_stages, stage_base_fn, gather_fn, out_plane_fn,
              dummy_hbm, edges_il, out, idx_v, rows_v, acc, gsems, ssems,
              feat, dtype):
    c = lax.axis_index("c")
    s = lax.axis_index("s")
    row0 = s * ROWS_PER_TILE

    def zfill(i, _):
        rows_v[pl.ds(i * 16, 16), :] = jnp.zeros((16, feat), dtype)
        return 0
    lax.fori_loop(0, GRP // 16, zfill, 0)
    def zero_acc(k, _):
        pltpu.sync_copy(rows_v.at[pl.ds(0, GRP), :],
                        acc.at[pl.ds(row0 + k * GRP, GRP), :])
        return 0
    lax.fori_loop(0, ROWS_PER_TILE // GRP, zero_acc, 0)
    pltpu.sync_copy(rows_v.at[pl.ds(0, CH), :],
                    acc.at[pl.ds(row0 + (ROWS_PER_TILE // GRP) * GRP,
                                 ROWS_PER_TILE - (ROWS_PER_TILE // GRP) * GRP),
                           :])
    plsc.subcore_barrier()

    def load_idx(st, g):
        n = stage_base_fn(c, s, st)
        pltpu.sync_copy(edges_il.at[n], idx_v.at[g])

    def drain_group(g, sems):
        pltpu.make_async_copy(dummy_hbm.at[pl.ds(0, GRP), :],
                              rows_v.at[pl.ds(g * GRP, GRP), :],
                              sems[g]).wait()

    def fire_scatters(g):
        pltpu.async_copy(rows_v.at[pl.ds(g * GRP, GRP), :],
                         acc.at[idx_v.at[g, 1]], ssems[g], add=True)

    n_pairs = n_stages // 2
    load_idx(0, 0)
    gather_fn(0)

    def stage_pair(p, _):
        e = 2 * p

        @pl.when(p > 0)
        def _():
            drain_group(1, ssems)
        load_idx(e + 1, 1)
        gather_fn(1)
        drain_group(0, gsems)
        fire_scatters(0)

        @pl.when(p + 1 < n_pairs)
        def _():
            drain_group(0, ssems)
            load_idx(e + 2, 0)
            gather_fn(0)
        drain_group(1, gsems)
        fire_scatters(1)
        return 0

    lax.fori_loop(0, n_pairs, stage_pair, 0)
    drain_group(0, ssems)
    drain_group(1, ssems)
    plsc.subcore_barrier()

    out_pl = out_plane_fn(out, c)
    n_out = ROWS_PER_TILE // GRP
    tail = ROWS_PER_TILE - n_out * GRP

    def out_cp(k2, _):
        for g in (0, 1):
            r = row0 + (2 * k2 + g) * GRP

            @pl.when(k2 > 0)
            def _(g=g):
                drain_group(g, gsems)
            pltpu.sync_copy(acc.at[pl.ds(r, GRP), :],
                            rows_v.at[pl.ds(g * GRP, GRP), :])
            pltpu.async_copy(rows_v.at[pl.ds(g * GRP, GRP), :],
                             out_pl.at[pl.ds(r, GRP), :], gsems[g])
        return 0
    lax.fori_loop(0, n_out // 2, out_cp, 0)
    rt = row0 + n_out * GRP
    gt = n_out % 2
    drain_group(gt, gsems)
    pltpu.sync_copy(acc.at[pl.ds(rt, tail), :],
                    rows_v.at[pl.ds(gt * GRP, tail), :])
    pltpu.async_copy(rows_v.at[pl.ds(gt * GRP, tail), :],
                     out_pl.at[pl.ds(rt, tail), :], gsems[gt])
    pltpu.make_async_copy(dummy_hbm.at[pl.ds(0, tail), :],
                          rows_v.at[pl.ds(gt * GRP, tail), :],
                          gsems[gt]).wait()
    pltpu.make_async_copy(dummy_hbm.at[pl.ds(0, GRP), :],
                          rows_v.at[pl.ds((1 - gt) * GRP, GRP), :],
                          gsems[1 - gt]).wait()


_SC_SCRATCH = [
    pltpu.VMEM((2, 2, GRP), jnp.int32),
    pltpu.VMEM((2 * GRP, F), jnp.float32),
    pltpu.VMEM_SHARED((NP, F), jnp.float32),
    pltpu.SemaphoreType.DMA,
    pltpu.SemaphoreType.DMA,
    pltpu.SemaphoreType.DMA,
    pltpu.SemaphoreType.DMA,
]


def _sc_mesh():
    return plsc.VectorSubcoreMesh(core_axis_name="c", subcore_axis_name="s",
                                  num_cores=NC, num_subcores=NS)


def _make_agg1():
    def stage_base(c, s, st):
        return (s * NC + c) * L1_STAGES + st

    def body(table, edges_il, out, idx_v, rows_v, acc,
             gsem0, gsem1, ssem0, ssem1):
        gsems = (gsem0, gsem1)

        def gather_fn(g):
            pltpu.async_copy(table.at[idx_v.at[g, 0]],
                             rows_v.at[pl.ds(g * GRP, GRP), :], gsems[g])

        _agg_body(L1_STAGES, stage_base, gather_fn,
                  lambda out, c: out.at[c],
                  table, edges_il, out, idx_v, rows_v, acc,
                  gsems, (ssem0, ssem1), F, jnp.float32)

    return pl.kernel(
        body,
        out_type=jax.ShapeDtypeStruct((NC, NP, F), jnp.float32),
        mesh=_sc_mesh(),
        scratch_types=list(_SC_SCRATCH),
        compiler_params=pltpu.CompilerParams(use_tc_tiling_on_sc=False),
    )


def _make_agg2():
    def stage_base(c, s, st):
        return (s * NC + c) * L1_STAGES + st

    def body(table, edges_il, out, idx_v, rows_v, acc,
             gsem0, gsem1, ssem0, ssem1):
        gsems = (gsem0, gsem1)

        def gather_fn(g):
            pltpu.async_copy(table.at[idx_v.at[g, 0]],
                             rows_v.at[pl.ds(g * GRP, GRP), :], gsems[g])

        _agg_body(L1_STAGES, stage_base, gather_fn,
                  lambda out, c: out.at[c],
                  table, edges_il, out, idx_v, rows_v, acc,
                  gsems, (ssem0, ssem1), F2, jnp.bfloat16)

    return pl.kernel(
        body,
        out_type=jax.ShapeDtypeStruct((NC, NP, F2), jnp.bfloat16),
        mesh=_sc_mesh(),
        scratch_types=[
            pltpu.VMEM((2, 2, GRP), jnp.int32),
            pltpu.VMEM((2 * GRP, F2), jnp.bfloat16),
            pltpu.VMEM_SHARED((NP, F2), jnp.bfloat16),
            pltpu.SemaphoreType.DMA,
            pltpu.SemaphoreType.DMA,
            pltpu.SemaphoreType.DMA,
            pltpu.SemaphoreType.DMA,
        ],
        compiler_params=pltpu.CompilerParams(use_tc_tiling_on_sc=False),
    )


PB = BLK // 8
PB4 = BLK // 4


def _layer1_block(p_ref, x_ref, w_rel_ref, w_root_ref, m_ref, b_ref, g_ref,
                  be_ref, out_ref):
    i = pl.program_id(0)
    agg = p_ref[0] + p_ref[1]
    xb = x_ref[...]
    dn = (((1,), (0,)), ((), ()))
    m = m_ref[...]
    halves = []
    for half in (0, 1):
        h = (lax.dot_general(agg, w_rel_ref[half], dn,
                             preferred_element_type=jnp.float32)
             + lax.dot_general(xb, w_root_ref[half], dn,
                               preferred_element_type=jnp.float32)
             + b_ref[...])
        mu = lax.dot_general(h, m, dn, preferred_element_type=jnp.float32)
        msq = lax.dot_general(h * h, m, dn,
                              preferred_element_type=jnp.float32)
        var = msq - mu * mu
        h = (h - mu) * lax.rsqrt(var + 1e-5) * g_ref[...] + be_ref[...]
        h = jnp.maximum(h, 0.0)
        r = lax.broadcasted_iota(jnp.int32, (PB, 128), 0)
        l = lax.broadcasted_iota(jnp.int32, (PB, 128), 1)
        nid = i * BLK + 8 * r + 4 * half + l // 32
        h = jnp.where(nid < N, h, 0.0)
        halves.append(h.astype(jnp.bfloat16))
    stacked = jnp.concatenate([halves[0][:, None, :], halves[1][:, None, :]],
                              axis=1)
    out_ref[...] = stacked.reshape(PB * 2, 128)


def _layer2_block(a_ref, h_ref, batch_ref, w_rel_ref, w_root_ref, m_ref,
                  b_ref, g_ref, be_ref, out_ref, acc_ref):
    i = pl.program_id(0)
    agg = (a_ref[0].astype(jnp.float32) + a_ref[1].astype(jnp.float32))
    hb = h_ref[...].astype(jnp.float32)
    dn = (((1,), (0,)), ((), ()))
    m = m_ref[...]

    @pl.when(i == 0)
    def _():
        acc_ref[...] = jnp.zeros_like(acc_ref)

    for half in (0, 1):
        h2 = (lax.dot_general(agg, w_rel_ref[half], dn,
                              preferred_element_type=jnp.float32)
              + lax.dot_general(hb, w_root_ref[half], dn,
                                preferred_element_type=jnp.float32)
              + b_ref[...])
        mu = lax.dot_general(h2, m, dn, preferred_element_type=jnp.float32)
        msq = lax.dot_general(h2 * h2, m, dn,
                              preferred_element_type=jnp.float32)
        var = msq - mu * mu
        h2 = (h2 - mu) * lax.rsqrt(var + 1e-5) * g_ref[...] + be_ref[...]
        h2 = jnp.maximum(h2, 0.0)
        for k in (0, 1):
            hseg = h2[:, k * G:(k + 1) * G]
            bk = batch_ref[0, 2 * half + k, :]
            onehot = (bk[:, None]
                      == lax.broadcasted_iota(jnp.int32, (PB4, G), 1))
            onehot = onehot.astype(jnp.float32)
            aug = jnp.concatenate([hseg, jnp.ones((PB4, G), jnp.float32)],
                                  axis=1)
            acc_ref[...] += lax.dot_general(
                onehot, aug, (((0,), (0,)), ((), ())),
                preferred_element_type=jnp.float32)

    @pl.when(i == NBLK - 1)
    def _():
        a = acc_ref[...]
        out_ref[...] = a[:, :G] / jnp.maximum(a[:, G:], 1.0)


_layer1_call = pl.pallas_call(
    _layer1_block,
    grid=(NBLK,),
    in_specs=[
        pl.BlockSpec((NC, PB, 128), lambda i: (0, i, 0)),
        pl.BlockSpec((PB, 128), lambda i: (i, 0)),
        pl.BlockSpec((2, 128, 128), lambda i: (0, 0, 0)),
        pl.BlockSpec((2, 128, 128), lambda i: (0, 0, 0)),
        pl.BlockSpec((128, 128), lambda i: (0, 0)),
        pl.BlockSpec((1, 128), lambda i: (0, 0)),
        pl.BlockSpec((1, 128), lambda i: (0, 0)),
        pl.BlockSpec((1, 128), lambda i: (0, 0)),
    ],
    out_specs=pl.BlockSpec((PB4, 128), lambda i: (i, 0)),
    out_shape=jax.ShapeDtypeStruct((NP // 4, 128), jnp.bfloat16),
)

_layer2_call = pl.pallas_call(
    _layer2_block,
    grid=(NBLK,),
    in_specs=[
        pl.BlockSpec((NC, PB4, 128), lambda i: (0, i, 0)),
        pl.BlockSpec((PB4, 128), lambda i: (i, 0)),
        pl.BlockSpec((1, 8, PB4), lambda i: (i, 0, 0)),
        pl.BlockSpec((2, 128, 128), lambda i: (0, 0, 0)),
        pl.BlockSpec((2, 128, 128), lambda i: (0, 0, 0)),
        pl.BlockSpec((128, 128), lambda i: (0, 0)),
        pl.BlockSpec((1, 128), lambda i: (0, 0)),
        pl.BlockSpec((1, 128), lambda i: (0, 0)),
        pl.BlockSpec((1, 128), lambda i: (0, 0)),
    ],
    out_specs=pl.BlockSpec((G, G), lambda i: (0, 0)),
    out_shape=jax.ShapeDtypeStruct((G, G), jnp.float32),
    scratch_shapes=[pltpu.VMEM((G, 2 * G), jnp.float32)],
)


def _block_diag(w_t, n_rep, half_out):
    fin, fout = w_t.shape
    eye = jnp.eye(n_rep, dtype=jnp.float32)
    big = (eye[:, None, :, None] * w_t[None, :, None, :]).reshape(
        n_rep * fin, n_rep * fout)
    return jnp.stack([big[:, :half_out], big[:, half_out:]])


def _seg_mean(block):
    lane = jnp.arange(128)
    return jnp.where(lane[:, None] // block == lane[None, :] // block,
                     1.0 / block, 0.0).astype(jnp.float32)


def kernel(x, edge_index, batch, W1_rel, b1, W1_root, ln1_g, ln1_b,
           W2_rel, b2, W2_root, ln2_g, ln2_b):
    x_lin = jnp.pad(x, ((0, NP - N), (0, F - D))).reshape(NP // 8, 128)
    src = edge_index[0]
    dst = edge_index[1]
    src_p = jnp.concatenate([src, jnp.full((EP - E,), N, jnp.int32)])
    dst_p = jnp.concatenate([dst, jnp.zeros((EP - E,), jnp.int32)])
    edges_il = jnp.stack([src_p.reshape(EP // GRP, GRP),
                          dst_p.reshape(EP // GRP, GRP)], axis=1)

    w1r_t = jnp.zeros((F, 32), jnp.float32).at[:D, :].set(W1_rel.T)
    w1o_t = jnp.zeros((F, 32), jnp.float32).at[:D, :].set(W1_root.T)
    w1r_big = _block_diag(w1r_t, 8, 128)
    w1o_big = _block_diag(w1o_t, 8, 128)
    w2r_big = _block_diag(W2_rel.T, 4, 128)
    w2o_big = _block_diag(W2_root.T, 4, 128)

    p1 = _make_agg1()(x_lin.reshape(NP, F), edges_il)
    h_pk = _layer1_call(p1.reshape(NC, NP // 8, 128), x_lin,
                        w1r_big, w1o_big, _seg_mean(32),
                        jnp.tile(b1, 4).reshape(1, 128),
                        jnp.tile(ln1_g, 4).reshape(1, 128),
                        jnp.tile(ln1_b, 4).reshape(1, 128))
    p2 = _make_agg2()(h_pk.reshape(NP, F2), edges_il)

    batch_p = jnp.concatenate([batch, jnp.full((NP - N,), G, jnp.int32)])
    batch4 = batch_p.reshape(NBLK, PB4, 4).transpose(0, 2, 1)
    batch8 = jnp.concatenate([batch4, batch4], axis=1)
    out = _layer2_call(p2.reshape(NC, NP // 4, 128), h_pk, batch8,
                       w2r_big, w2o_big, _seg_mean(G),
                       jnp.tile(b2, 2).reshape(1, 128),
                       jnp.tile(ln2_g, 2).reshape(1, 128),
                       jnp.tile(ln2_b, 2).reshape(1, 128))
    return out


# speedup vs baseline: 22.1572x; 1.0012x over previous
import functools

import jax
import jax.numpy as jnp
from jax import lax
from jax.experimental import pallas as pl
from jax.experimental.pallas import tpu as pltpu
from jax.experimental.pallas import tpu_sc as plsc

N = 100000
E = 1600000
D = 9
G = 64

NC = 2
NS = 16
F = 16
F2 = 32

NP = 100352
EP = 1605632
CH = 128
IB = 4
ROWS_PER_TILE = NP // NS
L1_STAGES = EP // (NC * NS) // (CH * IB)
BLK = 1024
NBLK = NP // BLK


GRP = IB * CH


def _agg_body(n_stages, stage_base_fn, gather_fn, out_plane_fn,
              dummy_hbm, src_hbm, dst_hbm, out, idx_v, rows_v, acc,
              gsems, ssems, isems, feat, dtype):
    c = lax.axis_index("c")
    s = lax.axis_index("s")
    row0 = s * ROWS_PER_TILE

    def zfill(i, _):
        rows_v[pl.ds(i * 16, 16), :] = jnp.zeros((16, feat), dtype)
        return 0
    lax.fori_loop(0, GRP // 16, zfill, 0)
    def zero_acc(k, _):
        pltpu.sync_copy(rows_v.at[pl.ds(0, GRP), :],
                        acc.at[pl.ds(row0 + k * GRP, GRP), :])
        return 0
    lax.fori_loop(0, ROWS_PER_TILE // GRP, zero_acc, 0)
    pltpu.sync_copy(rows_v.at[pl.ds(0, CH), :],
                    acc.at[pl.ds(row0 + (ROWS_PER_TILE // GRP) * GRP,
                                 ROWS_PER_TILE - (ROWS_PER_TILE // GRP) * GRP),
                           :])
    plsc.subcore_barrier()

    def load_idx(st, g):
        n = stage_base_fn(c, s, st) * GRP
        pltpu.async_copy(src_hbm.at[pl.ds(n, GRP)], idx_v.at[g, 0], isems[g])
        pltpu.async_copy(dst_hbm.at[pl.ds(n, GRP)], idx_v.at[g, 1], isems[g])
        pltpu.make_async_copy(src_hbm.at[pl.ds(0, GRP)], idx_v.at[g, 0],
                              isems[g]).wait()
        pltpu.make_async_copy(src_hbm.at[pl.ds(0, GRP)], idx_v.at[g, 1],
                              isems[g]).wait()

    def drain_group(g, sems):
        pltpu.make_async_copy(dummy_hbm.at[pl.ds(0, GRP), :],
                              rows_v.at[pl.ds(g * GRP, GRP), :],
                              sems[g]).wait()

    def fire_scatters(g):
        pltpu.async_copy(rows_v.at[pl.ds(g * GRP, GRP), :],
                         acc.at[idx_v.at[g, 1]], ssems[g], add=True)

    n_pairs = n_stages // 2
    load_idx(0, 0)
    gather_fn(0)

    def stage_pair(p, _):
        e = 2 * p

        @pl.when(p > 0)
        def _():
            drain_group(1, ssems)
        load_idx(e + 1, 1)
        gather_fn(1)
        drain_group(0, gsems)
        fire_scatters(0)

        @pl.when(p + 1 < n_pairs)
        def _():
            drain_group(0, ssems)
            load_idx(e + 2, 0)
            gather_fn(0)
        drain_group(1, gsems)
        fire_scatters(1)
        return 0

    lax.fori_loop(0, n_pairs, stage_pair, 0)
    drain_group(0, ssems)
    drain_group(1, ssems)
    plsc.subcore_barrier()

    out_pl = out_plane_fn(out, c)
    n_out = ROWS_PER_TILE // GRP
    tail = ROWS_PER_TILE - n_out * GRP

    def out_cp(k2, _):
        for g in (0, 1):
            r = row0 + (2 * k2 + g) * GRP

            @pl.when(k2 > 0)
            def _(g=g):
                drain_group(g, gsems)
            pltpu.sync_copy(acc.at[pl.ds(r, GRP), :],
                            rows_v.at[pl.ds(g * GRP, GRP), :])
            pltpu.async_copy(rows_v.at[pl.ds(g * GRP, GRP), :],
                             out_pl.at[pl.ds(r, GRP), :], gsems[g])
        return 0
    lax.fori_loop(0, n_out // 2, out_cp, 0)
    rt = row0 + n_out * GRP
    gt = n_out % 2
    drain_group(gt, gsems)
    pltpu.sync_copy(acc.at[pl.ds(rt, tail), :],
                    rows_v.at[pl.ds(gt * GRP, tail), :])
    pltpu.async_copy(rows_v.at[pl.ds(gt * GRP, tail), :],
                     out_pl.at[pl.ds(rt, tail), :], gsems[gt])
    pltpu.make_async_copy(dummy_hbm.at[pl.ds(0, tail), :],
                          rows_v.at[pl.ds(gt * GRP, tail), :],
                          gsems[gt]).wait()
    pltpu.make_async_copy(dummy_hbm.at[pl.ds(0, GRP), :],
                          rows_v.at[pl.ds((1 - gt) * GRP, GRP), :],
                          gsems[1 - gt]).wait()


_SC_SCRATCH = [
    pltpu.VMEM((2, 2, GRP), jnp.int32),
    pltpu.VMEM((2 * GRP, F), jnp.float32),
    pltpu.VMEM_SHARED((NP, F), jnp.float32),
    pltpu.SemaphoreType.DMA,
    pltpu.SemaphoreType.DMA,
    pltpu.SemaphoreType.DMA,
    pltpu.SemaphoreType.DMA,
    pltpu.SemaphoreType.DMA,
    pltpu.SemaphoreType.DMA,
]


def _sc_mesh():
    return plsc.VectorSubcoreMesh(core_axis_name="c", subcore_axis_name="s",
                                  num_cores=NC, num_subcores=NS)


def _make_agg1():
    def stage_base(c, s, st):
        return (s * NC + c) * L1_STAGES + st

    def body(table, src_hbm, dst_hbm, out, idx_v, rows_v, acc,
             gsem0, gsem1, ssem0, ssem1, isem0, isem1):
        gsems = (gsem0, gsem1)

        def gather_fn(g):
            pltpu.async_copy(table.at[idx_v.at[g, 0]],
                             rows_v.at[pl.ds(g * GRP, GRP), :], gsems[g])

        _agg_body(L1_STAGES, stage_base, gather_fn,
                  lambda out, c: out.at[c],
                  table, src_hbm, dst_hbm, out, idx_v, rows_v, acc,
                  gsems, (ssem0, ssem1), (isem0, isem1), F, jnp.float32)

    return pl.kernel(
        body,
        out_type=jax.ShapeDtypeStruct((NC, NP, F), jnp.float32),
        mesh=_sc_mesh(),
        scratch_types=list(_SC_SCRATCH),
        compiler_params=pltpu.CompilerParams(use_tc_tiling_on_sc=False),
    )


def _make_agg2():
    def stage_base(c, s, st):
        return (s * NC + c) * L1_STAGES + st

    def body(table, src_hbm, dst_hbm, out, idx_v, rows_v, acc,
             gsem0, gsem1, ssem0, ssem1, isem0, isem1):
        gsems = (gsem0, gsem1)

        def gather_fn(g):
            pltpu.async_copy(table.at[idx_v.at[g, 0]],
                             rows_v.at[pl.ds(g * GRP, GRP), :], gsems[g])

        _agg_body(L1_STAGES, stage_base, gather_fn,
                  lambda out, c: out.at[c],
                  table, src_hbm, dst_hbm, out, idx_v, rows_v, acc,
                  gsems, (ssem0, ssem1), (isem0, isem1), F2, jnp.bfloat16)

    return pl.kernel(
        body,
        out_type=jax.ShapeDtypeStruct((NC, NP, F2), jnp.bfloat16),
        mesh=_sc_mesh(),
        scratch_types=[
            pltpu.VMEM((2, 2, GRP), jnp.int32),
            pltpu.VMEM((2 * GRP, F2), jnp.bfloat16),
            pltpu.VMEM_SHARED((NP, F2), jnp.bfloat16),
            pltpu.SemaphoreType.DMA,
            pltpu.SemaphoreType.DMA,
            pltpu.SemaphoreType.DMA,
            pltpu.SemaphoreType.DMA,
            pltpu.SemaphoreType.DMA,
            pltpu.SemaphoreType.DMA,
        ],
        compiler_params=pltpu.CompilerParams(use_tc_tiling_on_sc=False),
    )


PB = BLK // 8
PB4 = BLK // 4


def _layer1_block(p_ref, x_ref, w_rel_ref, w_root_ref, m_ref, b_ref, g_ref,
                  be_ref, out_ref):
    i = pl.program_id(0)
    agg = p_ref[0] + p_ref[1]
    xb = x_ref[...]
    dn = (((1,), (0,)), ((), ()))
    m = m_ref[...]
    halves = []
    for half in (0, 1):
        h = (lax.dot_general(agg, w_rel_ref[half], dn,
                             preferred_element_type=jnp.float32)
             + lax.dot_general(xb, w_root_ref[half], dn,
                               preferred_element_type=jnp.float32)
             + b_ref[...])
        mu = lax.dot_general(h, m, dn, preferred_element_type=jnp.float32)
        msq = lax.dot_general(h * h, m, dn,
                              preferred_element_type=jnp.float32)
        var = msq - mu * mu
        h = (h - mu) * lax.rsqrt(var + 1e-5) * g_ref[...] + be_ref[...]
        h = jnp.maximum(h, 0.0)
        r = lax.broadcasted_iota(jnp.int32, (PB, 128), 0)
        l = lax.broadcasted_iota(jnp.int32, (PB, 128), 1)
        nid = i * BLK + 8 * r + 4 * half + l // 32
        h = jnp.where(nid < N, h, 0.0)
        halves.append(h.astype(jnp.bfloat16))
    stacked = jnp.concatenate([halves[0][:, None, :], halves[1][:, None, :]],
                              axis=1)
    out_ref[...] = stacked.reshape(PB * 2, 128)


def _layer2_block(a_ref, h_ref, batch_ref, w_rel_ref, w_root_ref, m_ref,
                  b_ref, g_ref, be_ref, out_ref, acc_ref):
    i = pl.program_id(0)
    agg = (a_ref[0].astype(jnp.float32) + a_ref[1].astype(jnp.float32))
    hb = h_ref[...].astype(jnp.float32)
    dn = (((1,), (0,)), ((), ()))
    m = m_ref[...]

    @pl.when(i == 0)
    def _():
        acc_ref[...] = jnp.zeros_like(acc_ref)

    for half in (0, 1):
        h2 = (lax.dot_general(agg, w_rel_ref[half], dn,
                              preferred_element_type=jnp.float32)
              + lax.dot_general(hb, w_root_ref[half], dn,
                                preferred_element_type=jnp.float32)
              + b_ref[...])
        mu = lax.dot_general(h2, m, dn, preferred_element_type=jnp.float32)
        msq = lax.dot_general(h2 * h2, m, dn,
                              preferred_element_type=jnp.float32)
        var = msq - mu * mu
        h2 = (h2 - mu) * lax.rsqrt(var + 1e-5) * g_ref[...] + be_ref[...]
        h2 = jnp.maximum(h2, 0.0)
        for k in (0, 1):
            hseg = h2[:, k * G:(k + 1) * G]
            bk = batch_ref[0, 2 * half + k, :]
            onehot = (bk[:, None]
                      == lax.broadcasted_iota(jnp.int32, (PB4, G), 1))
            onehot = onehot.astype(jnp.float32)
            aug = jnp.concatenate([hseg, jnp.ones((PB4, G), jnp.float32)],
                                  axis=1)
            acc_ref[...] += lax.dot_general(
                onehot, aug, (((0,), (0,)), ((), ())),
                preferred_element_type=jnp.float32)

    @pl.when(i == NBLK - 1)
    def _():
        a = acc_ref[...]
        out_ref[...] = a[:, :G] / jnp.maximum(a[:, G:], 1.0)


_layer1_call = pl.pallas_call(
    _layer1_block,
    grid=(NBLK,),
    in_specs=[
        pl.BlockSpec((NC, PB, 128), lambda i: (0, i, 0)),
        pl.BlockSpec((PB, 128), lambda i: (i, 0)),
        pl.BlockSpec((2, 128, 128), lambda i: (0, 0, 0)),
        pl.BlockSpec((2, 128, 128), lambda i: (0, 0, 0)),
        pl.BlockSpec((128, 128), lambda i: (0, 0)),
        pl.BlockSpec((1, 128), lambda i: (0, 0)),
        pl.BlockSpec((1, 128), lambda i: (0, 0)),
        pl.BlockSpec((1, 128), lambda i: (0, 0)),
    ],
    out_specs=pl.BlockSpec((PB4, 128), lambda i: (i, 0)),
    out_shape=jax.ShapeDtypeStruct((NP // 4, 128), jnp.bfloat16),
)

_layer2_call = pl.pallas_call(
    _layer2_block,
    grid=(NBLK,),
    in_specs=[
        pl.BlockSpec((NC, PB4, 128), lambda i: (0, i, 0)),
        pl.BlockSpec((PB4, 128), lambda i: (i, 0)),
        pl.BlockSpec((1, 8, PB4), lambda i: (i, 0, 0)),
        pl.BlockSpec((2, 128, 128), lambda i: (0, 0, 0)),
        pl.BlockSpec((2, 128, 128), lambda i: (0, 0, 0)),
        pl.BlockSpec((128, 128), lambda i: (0, 0)),
        pl.BlockSpec((1, 128), lambda i: (0, 0)),
        pl.BlockSpec((1, 128), lambda i: (0, 0)),
        pl.BlockSpec((1, 128), lambda i: (0, 0)),
    ],
    out_specs=pl.BlockSpec((G, G), lambda i: (0, 0)),
    out_shape=jax.ShapeDtypeStruct((G, G), jnp.float32),
    scratch_shapes=[pltpu.VMEM((G, 2 * G), jnp.float32)],
)


def _block_diag(w_t, n_rep, half_out):
    fin, fout = w_t.shape
    eye = jnp.eye(n_rep, dtype=jnp.float32)
    big = (eye[:, None, :, None] * w_t[None, :, None, :]).reshape(
        n_rep * fin, n_rep * fout)
    return jnp.stack([big[:, :half_out], big[:, half_out:]])


def _seg_mean(block):
    lane = jnp.arange(128)
    return jnp.where(lane[:, None] // block == lane[None, :] // block,
                     1.0 / block, 0.0).astype(jnp.float32)


def kernel(x, edge_index, batch, W1_rel, b1, W1_root, ln1_g, ln1_b,
           W2_rel, b2, W2_root, ln2_g, ln2_b):
    x_lin = jnp.pad(x, ((0, NP - N), (0, F - D))).reshape(NP // 8, 128)
    src = edge_index[0]
    dst = edge_index[1]
    src_p = jnp.concatenate([src, jnp.full((EP - E,), N, jnp.int32)])
    dst_p = jnp.concatenate([dst, jnp.zeros((EP - E,), jnp.int32)])
    w1r_t = jnp.zeros((F, 32), jnp.float32).at[:D, :].set(W1_rel.T)
    w1o_t = jnp.zeros((F, 32), jnp.float32).at[:D, :].set(W1_root.T)
    w1r_big = _block_diag(w1r_t, 8, 128)
    w1o_big = _block_diag(w1o_t, 8, 128)
    w2r_big = _block_diag(W2_rel.T, 4, 128)
    w2o_big = _block_diag(W2_root.T, 4, 128)

    p1 = _make_agg1()(x_lin.reshape(NP, F), src_p, dst_p)
    h_pk = _layer1_call(p1.reshape(NC, NP // 8, 128), x_lin,
                        w1r_big, w1o_big, _seg_mean(32),
                        jnp.tile(b1, 4).reshape(1, 128),
                        jnp.tile(ln1_g, 4).reshape(1, 128),
                        jnp.tile(ln1_b, 4).reshape(1, 128))
    p2 = _make_agg2()(h_pk.reshape(NP, F2), src_p, dst_p)

    batch_p = jnp.concatenate([batch, jnp.full((NP - N,), G, jnp.int32)])
    batch4 = batch_p.reshape(NBLK, PB4, 4).transpose(0, 2, 1)
    batch8 = jnp.concatenate([batch4, batch4], axis=1)
    out = _layer2_call(p2.reshape(NC, NP // 4, 128), h_pk, batch8,
                       w2r_big, w2o_big, _seg_mean(G),
                       jnp.tile(b2, 2).reshape(1, 128),
                       jnp.tile(ln2_g, 2).reshape(1, 128),
                       jnp.tile(ln2_b, 2).reshape(1, 128))
    return out
